# baseline (speedup 1.0000x reference)
import functools

import jax
import jax.numpy as jnp
from jax import lax
from jax.experimental import pallas as pl
from jax.experimental.pallas import tpu as pltpu
from jax.experimental.pallas import tpu_sc as plsc

_NUM_TILES = 32
_BLK_SAMP = 16


def _proj_body(x_ref, wt_ref, o_ref):
    o_ref[...] = jnp.dot(x_ref[...], wt_ref[...],
                         preferred_element_type=jnp.float32)


def _project_table(t16, wtb):
    g = t16.shape[0]
    blk = 1960
    return pl.pallas_call(
        _proj_body,
        grid=(g // blk,),
        in_specs=[
            pl.BlockSpec((blk, 1024), lambda i: (i, 0)),
            pl.BlockSpec((1024, 128), lambda i: (0, 0)),
        ],
        out_specs=pl.BlockSpec((blk, 128), lambda i: (i, 0)),
        out_shape=jax.ShapeDtypeStruct((g, 128), jnp.float32),
    )(t16, wtb)


def _make_sc_pool(batch, seq):
    n_blocks = batch // _BLK_SAMP
    blocks_per_tile = n_blocks // _NUM_TILES
    out_words_per_tile = blocks_per_tile * _BLK_SAMP * 2
    mesh = plsc.VectorSubcoreMesh(core_axis_name="c", subcore_axis_name="s",
                                  num_cores=2, num_subcores=16)

    @functools.partial(
        pl.kernel,
        out_type=jax.ShapeDtypeStruct((batch * 2,), jnp.float32),
        mesh=mesh,
        scratch_types=[
            pltpu.VMEM((seq * _BLK_SAMP,), jnp.int32),
            pltpu.VMEM((seq * _BLK_SAMP, 8), jnp.float32),
            pltpu.VMEM((out_words_per_tile,), jnp.float32),
            pltpu.VMEM((16,), jnp.float32),
            pltpu.VMEM((16,), jnp.int32),
            pltpu.SemaphoreType.DMA,
        ],
        compiler_params=pltpu.CompilerParams(
            needs_layout_passes=False, use_tc_tiling_on_sc=False),
    )
    def sc_pool(tok_hbm, proj_hbm, bpair_hbm, out_hbm,
                tok_v, rows_v, out_v, b_v, cnt_v, sem):
        wid = lax.axis_index("s") * 2 + lax.axis_index("c")
        pltpu.sync_copy(bpair_hbm, b_v)
        b_pair = b_v[...]

        iota = lax.iota(jnp.int32, 16)
        s_lo = lax.shift_right_logical(iota, 1)
        s_hi = s_lo + 8
        d_v = jnp.bitwise_and(iota, 1)
        zero_f = jnp.zeros((16,), jnp.float32)
        zero_i = jnp.zeros((16,), jnp.int32)

        def block_body(j, carry):
            bid = wid * blocks_per_tile + j
            pltpu.sync_copy(tok_hbm.at[bid], tok_v)
            pltpu.async_copy(proj_hbm.at[tok_v], rows_v, sem).wait()

            def tok_body(t, tc):
                a0, a1, c = tc
                base = jnp.full((16,), t * 16, jnp.int32)
                a0 = a0 + plsc.load_gather(rows_v, [base + s_lo, d_v])
                a1 = a1 + plsc.load_gather(rows_v, [base + s_hi, d_v])
                c = c + (tok_v[pl.ds(t * 16, 16)] != 0).astype(jnp.int32)
                return a0, a1, c

            a0, a1, c = lax.fori_loop(
                0, seq, tok_body, (zero_f, zero_f, zero_i))

            cnt_v[...] = c
            c0 = plsc.load_gather(cnt_v, [s_lo])
            c1 = plsc.load_gather(cnt_v, [s_hi])
            cf0 = jnp.maximum(c0, 1).astype(jnp.float32)
            cf1 = jnp.maximum(c1, 1).astype(jnp.float32)
            out_v[pl.ds(j * 32, 16)] = a0 / cf0 + b_pair
            out_v[pl.ds(j * 32 + 16, 16)] = a1 / cf1 + b_pair
            return carry

        lax.fori_loop(0, blocks_per_tile, block_body, 0)
        pltpu.sync_copy(
            out_v, out_hbm.at[pl.ds(wid * out_words_per_tile,
                                    out_words_per_tile)])

    return sc_pool


def kernel(token_ids, table, W, b):
    batch, seq = token_ids.shape
    v = table.shape[0]
    vpad = 62720 * 16
    wtb = jnp.zeros((1024, 128), jnp.float32)
    for k in range(16):
        wtb = lax.dynamic_update_slice(wtb, W.T, (k * 64, k * 8))
    t16 = jnp.pad(table, ((0, vpad - v), (0, 0))).reshape(vpad // 16, 1024)
    proj = _project_table(t16, wtb).reshape(vpad, 8)
    tok_blocks = jnp.swapaxes(
        token_ids.reshape(batch // _BLK_SAMP, _BLK_SAMP, seq), 1, 2
    ).reshape(batch // _BLK_SAMP, seq * _BLK_SAMP)
    b_pair = jnp.tile(b.astype(jnp.float32), 8)
    out = _make_sc_pool(batch, seq)(tok_blocks, proj, b_pair)
    return out.reshape(batch, 2)


# speedup vs baseline: 1.0790x; 1.0790x over previous
import functools

import jax
import jax.numpy as jnp
from jax import lax
from jax.experimental import pallas as pl
from jax.experimental.pallas import tpu as pltpu
from jax.experimental.pallas import tpu_sc as plsc

_NUM_TILES = 32
_BLK_SAMP = 16


def _proj_body(x_ref, wt_ref, o_ref):
    o_ref[...] = jnp.dot(x_ref[...], wt_ref[...],
                         preferred_element_type=jnp.float32)


def _project_table(table, wt):
    v, d = table.shape
    blk = next(c for c in (10000, 8192, 8000, 6400, 5000, 4096, 4000,
                           2000, 1000, 8)
               if v % c == 0 and c % 8 == 0)
    return pl.pallas_call(
        _proj_body,
        grid=(v // blk,),
        in_specs=[
            pl.BlockSpec((blk, d), lambda i: (i, 0)),
            pl.BlockSpec((d, 8), lambda i: (0, 0)),
        ],
        out_specs=pl.BlockSpec((blk, 8), lambda i: (i, 0)),
        out_shape=jax.ShapeDtypeStruct((v, 8), jnp.float32),
    )(table, wt)


def _make_sc_pool(batch, seq):
    n_blocks = batch // _BLK_SAMP
    bpt = n_blocks // _NUM_TILES
    out_words = bpt * _BLK_SAMP * 2
    ntok = seq * _BLK_SAMP
    mesh = plsc.VectorSubcoreMesh(core_axis_name="c", subcore_axis_name="s",
                                  num_cores=2, num_subcores=16)

    @functools.partial(
        pl.kernel,
        out_type=jax.ShapeDtypeStruct((batch * 2,), jnp.float32),
        mesh=mesh,
        scratch_types=[
            pltpu.VMEM((ntok,), jnp.int32),
            pltpu.VMEM((ntok,), jnp.int32),
            pltpu.VMEM((ntok, 8), jnp.float32),
            pltpu.VMEM((ntok, 8), jnp.float32),
            pltpu.VMEM((out_words,), jnp.float32),
            pltpu.VMEM((16,), jnp.float32),
            pltpu.VMEM((16,), jnp.int32),
            pltpu.SemaphoreType.DMA,
            pltpu.SemaphoreType.DMA,
        ],
        compiler_params=pltpu.CompilerParams(
            needs_layout_passes=False, use_tc_tiling_on_sc=False),
    )
    def sc_pool(tok_hbm, proj_hbm, bpair_hbm, out_hbm,
                tok0, tok1, rows0, rows1, out_v, b_v, cnt_v, sem0, sem1):
        wid = lax.axis_index("s") * 2 + lax.axis_index("c")
        pltpu.sync_copy(bpair_hbm, b_v)
        b_pair = b_v[...]

        iota = lax.iota(jnp.int32, 16)
        s_lo = lax.shift_right_logical(iota, 1)
        s_hi = s_lo + 8
        d_v = jnp.bitwise_and(iota, 1)
        zero_f = jnp.zeros((16,), jnp.float32)
        zero_i = jnp.zeros((16,), jnp.int32)

        def stage(j, tok_n, rows_n, sem_n):
            pltpu.sync_copy(tok_hbm.at[wid * bpt + j], tok_n)
            pltpu.async_copy(proj_hbm.at[tok_n], rows_n, sem_n)

        def compute(j, tok_c, rows_c):
            def tok_body(t, tc):
                a0, a1, c = tc
                base = jnp.full((16,), t * 16, jnp.int32)
                a0 = a0 + plsc.load_gather(rows_c, [base + s_lo, d_v])
                a1 = a1 + plsc.load_gather(rows_c, [base + s_hi, d_v])
                c = c + (tok_c[pl.ds(t * 16, 16)] != 0).astype(jnp.int32)
                return a0, a1, c

            a0, a1, c = lax.fori_loop(
                0, seq, tok_body, (zero_f, zero_f, zero_i))
            cnt_v[...] = c
            c0 = plsc.load_gather(cnt_v, [s_lo])
            c1 = plsc.load_gather(cnt_v, [s_hi])
            cf0 = jnp.maximum(c0, 1).astype(jnp.float32)
            cf1 = jnp.maximum(c1, 1).astype(jnp.float32)
            out_v[pl.ds(j * 32, 16)] = a0 / cf0 + b_pair
            out_v[pl.ds(j * 32 + 16, 16)] = a1 / cf1 + b_pair

        stage(0, tok0, rows0, sem0)
        bufs = ((tok0, rows0, sem0), (tok1, rows1, sem1))

        def pair_body(j2, carry):
            for h in (0, 1):
                j = j2 * 2 + h
                tok_c, rows_c, sem_c = bufs[h]
                tok_n, rows_n, sem_n = bufs[1 - h]

                @pl.when(j + 1 < bpt)
                def _():
                    stage(j + 1, tok_n, rows_n, sem_n)

                pltpu.make_async_copy(
                    proj_hbm.at[tok_c], rows_c, sem_c).wait()
                compute(j, tok_c, rows_c)
            return carry

        lax.fori_loop(0, bpt // 2, pair_body, 0)
        pltpu.sync_copy(out_v, out_hbm.at[pl.ds(wid * out_words, out_words)])

    return sc_pool


def kernel(token_ids, table, W, b):
    batch, seq = token_ids.shape
    wt = jnp.zeros((W.shape[1], 8), jnp.float32).at[:, :2].set(W.T)
    proj = _project_table(table, wt)
    tok_blocks = jnp.swapaxes(
        token_ids.reshape(batch // _BLK_SAMP, _BLK_SAMP, seq), 1, 2
    ).reshape(batch // _BLK_SAMP, seq * _BLK_SAMP)
    b_pair = jnp.tile(b.astype(jnp.float32), 8)
    out = _make_sc_pool(batch, seq)(tok_blocks, proj, b_pair)
    return out.reshape(batch, 2)


# speedup vs baseline: 1.5882x; 1.4719x over previous
import functools

import jax
import jax.numpy as jnp
from jax import lax
from jax.experimental import pallas as pl
from jax.experimental.pallas import tpu as pltpu
from jax.experimental.pallas import tpu_sc as plsc

_NUM_TILES = 32
_BLK_SAMP = 16


def _proj_body(xt_ref, wt_ref, o_ref):
    o_ref[...] = lax.dot_general(
        xt_ref[...], wt_ref[...], (((0,), (0,)), ((), ())),
        preferred_element_type=jnp.float32)


def _project_table(table_t, wt):
    d, v = table_t.shape
    blk = 8192
    return pl.pallas_call(
        _proj_body,
        grid=(pl.cdiv(v, blk),),
        in_specs=[
            pl.BlockSpec((d, blk), lambda i: (0, i)),
            pl.BlockSpec((d, 8), lambda i: (0, 0)),
        ],
        out_specs=pl.BlockSpec((blk, 8), lambda i: (i, 0)),
        out_shape=jax.ShapeDtypeStruct((v, 8), jnp.float32),
    )(table_t, wt)


def _make_sc_pool(batch, seq):
    n_blocks = batch // _BLK_SAMP
    bpt = n_blocks // _NUM_TILES
    out_words = bpt * _BLK_SAMP * 2
    ntok = seq * _BLK_SAMP
    mesh = plsc.VectorSubcoreMesh(core_axis_name="c", subcore_axis_name="s",
                                  num_cores=2, num_subcores=16)

    @functools.partial(
        pl.kernel,
        out_type=jax.ShapeDtypeStruct((batch * 2,), jnp.float32),
        mesh=mesh,
        scratch_types=[
            pltpu.VMEM((ntok,), jnp.int32),
            pltpu.VMEM((ntok,), jnp.int32),
            pltpu.VMEM((ntok, 8), jnp.float32),
            pltpu.VMEM((ntok, 8), jnp.float32),
            pltpu.VMEM((out_words,), jnp.float32),
            pltpu.VMEM((16,), jnp.float32),
            pltpu.VMEM((16,), jnp.int32),
            pltpu.SemaphoreType.DMA,
            pltpu.SemaphoreType.DMA,
        ],
        compiler_params=pltpu.CompilerParams(
            needs_layout_passes=False, use_tc_tiling_on_sc=False),
    )
    def sc_pool(tok_hbm, proj_hbm, bpair_hbm, out_hbm,
                tok0, tok1, rows0, rows1, out_v, b_v, cnt_v, sem0, sem1):
        wid = lax.axis_index("s") * 2 + lax.axis_index("c")
        pltpu.sync_copy(bpair_hbm, b_v)
        b_pair = b_v[...]

        iota = lax.iota(jnp.int32, 16)
        s_lo = lax.shift_right_logical(iota, 1)
        s_hi = s_lo + 8
        d_v = jnp.bitwise_and(iota, 1)
        zero_f = jnp.zeros((16,), jnp.float32)
        zero_i = jnp.zeros((16,), jnp.int32)

        def stage(j, tok_n, rows_n, sem_n):
            pltpu.sync_copy(tok_hbm.at[wid * bpt + j], tok_n)
            pltpu.async_copy(proj_hbm.at[tok_n], rows_n, sem_n)

        def compute(j, tok_c, rows_c):
            def tok_body(t, tc):
                a0, a1, c = tc
                base = jnp.full((16,), t * 16, jnp.int32)
                a0 = a0 + plsc.load_gather(rows_c, [base + s_lo, d_v])
                a1 = a1 + plsc.load_gather(rows_c, [base + s_hi, d_v])
                c = c + (tok_c[pl.ds(t * 16, 16)] != 0).astype(jnp.int32)
                return a0, a1, c

            a0, a1, c = lax.fori_loop(
                0, seq, tok_body, (zero_f, zero_f, zero_i))
            cnt_v[...] = c
            c0 = plsc.load_gather(cnt_v, [s_lo])
            c1 = plsc.load_gather(cnt_v, [s_hi])
            cf0 = jnp.maximum(c0, 1).astype(jnp.float32)
            cf1 = jnp.maximum(c1, 1).astype(jnp.float32)
            out_v[pl.ds(j * 32, 16)] = a0 / cf0 + b_pair
            out_v[pl.ds(j * 32 + 16, 16)] = a1 / cf1 + b_pair

        stage(0, tok0, rows0, sem0)
        bufs = ((tok0, rows0, sem0), (tok1, rows1, sem1))

        def pair_body(j2, carry):
            for h in (0, 1):
                j = j2 * 2 + h
                tok_c, rows_c, sem_c = bufs[h]
                tok_n, rows_n, sem_n = bufs[1 - h]

                @pl.when(j + 1 < bpt)
                def _():
                    stage(j + 1, tok_n, rows_n, sem_n)

                pltpu.make_async_copy(
                    proj_hbm.at[tok_c], rows_c, sem_c).wait()
                compute(j, tok_c, rows_c)
            return carry

        lax.fori_loop(0, bpt // 2, pair_body, 0)
        pltpu.sync_copy(out_v, out_hbm.at[pl.ds(wid * out_words, out_words)])

    return sc_pool


def kernel(token_ids, table, W, b):
    batch, seq = token_ids.shape
    wt = jnp.zeros((W.shape[1], 8), jnp.float32).at[:, :2].set(W.T)
    proj = _project_table(jnp.swapaxes(table, 0, 1), wt)
    tok_blocks = jnp.swapaxes(
        token_ids.reshape(batch // _BLK_SAMP, _BLK_SAMP, seq), 1, 2
    ).reshape(batch // _BLK_SAMP, seq * _BLK_SAMP)
    b_pair = jnp.tile(b.astype(jnp.float32), 8)
    out = _make_sc_pool(batch, seq)(tok_blocks, proj, b_pair)
    return out.reshape(batch, 2)
